# baseline (speedup 1.0000x reference)
import functools
import numpy as np
import jax
import jax.numpy as jnp
from jax import lax
from jax.experimental import pallas as pl
from jax.experimental.pallas import tpu as pltpu, tpu_sc as plsc

D_MODEL = 64
L_SEQ = 200
PE_ELEMS = L_SEQ * D_MODEL

NUM_CORES = 2
NUM_SUBCORES = 16
NW = NUM_CORES * NUM_SUBCORES

CH = 128


def _positional_encoding_np(max_len, d_model):
    pos = np.expand_dims(np.arange(0, max_len), axis=1)
    div_term = np.array(
        [[1 / np.power(10000, 2 * (i // 2) / d_model) for i in range(d_model)]]
    )
    posd = pos * div_term
    pe = np.zeros((max_len, d_model))
    pe[:, 0 : d_model // 2] = np.sin(posd[:, 0::2])
    pe[:, d_model // 2 :] = np.cos(posd[:, 0::2])
    return pe.astype(np.float32)


def _make_embed(flat_n):
    per_w = flat_n // NW
    n_chunks = per_w // CH
    chunks_2d = flat_n // CH

    mesh = plsc.VectorSubcoreMesh(
        core_axis_name="c", subcore_axis_name="s",
        num_cores=NUM_CORES, num_subcores=NUM_SUBCORES,
    )

    @functools.partial(
        pl.kernel,
        out_type=jax.ShapeDtypeStruct((flat_n, D_MODEL), jnp.float32),
        mesh=mesh,
        scratch_types=[
            pltpu.VMEM((n_chunks, CH), jnp.int32),
            pltpu.VMEM((PE_ELEMS,), jnp.float32),
            pltpu.VMEM((CH, D_MODEL), jnp.float32),
            pltpu.SemaphoreType.DMA,
        ],
        compiler_params=pltpu.CompilerParams(use_tc_tiling_on_sc=False),
    )
    def embed(x_hbm, table_hbm, pe_hbm, out_hbm, idx_v, pe_v, rows_v, gsem):
        wid = lax.axis_index("s") * NUM_CORES + lax.axis_index("c")
        chunk0 = wid * n_chunks
        pltpu.sync_copy(x_hbm.at[pl.ds(chunk0, n_chunks)], idx_v)
        pltpu.sync_copy(pe_hbm, pe_v)

        def chunk_body(g, carry):
            row_base = (chunk0 + g) * CH
            pos0 = lax.rem(row_base, L_SEQ)
            pltpu.async_copy(table_hbm.at[idx_v.at[g]], rows_v, gsem).wait()

            def row_body(r, c2):
                pos = pos0 + r
                pos = jnp.where(pos >= L_SEQ, pos - L_SEQ, pos)
                pb = pos * D_MODEL
                for c in range(D_MODEL // 16):
                    rows_v[r, pl.ds(c * 16, 16)] = (
                        rows_v[r, pl.ds(c * 16, 16)] * 8.0
                        + pe_v[pl.ds(pb + c * 16, 16)]
                    )
                return c2

            lax.fori_loop(0, CH, row_body, 0, unroll=True)
            pltpu.sync_copy(rows_v, out_hbm.at[pl.ds(row_base, CH)])
            return carry

        lax.fori_loop(0, n_chunks, chunk_body, 0)

    return embed


def kernel(x, emb_table):
    b, l = x.shape
    flat_n = b * l
    assert l == L_SEQ and flat_n % (NW * CH) == 0
    pe = jnp.asarray(_positional_encoding_np(L_SEQ, D_MODEL).reshape(-1))
    x2 = x.reshape(flat_n // CH, CH).astype(jnp.int32)
    out = _make_embed(flat_n)(x2, emb_table, pe)
    return out.reshape(b, l, D_MODEL)


# speedup vs baseline: 1.3871x; 1.3871x over previous
import functools
import numpy as np
import jax
import jax.numpy as jnp
from jax import lax
from jax.experimental import pallas as pl
from jax.experimental.pallas import tpu as pltpu, tpu_sc as plsc

D_MODEL = 64
L_SEQ = 200
PE_ELEMS = L_SEQ * D_MODEL

NUM_CORES = 2
NUM_SUBCORES = 16
NW = NUM_CORES * NUM_SUBCORES

CH = 128


def _positional_encoding_np(max_len, d_model):
    pos = np.expand_dims(np.arange(0, max_len), axis=1)
    div_term = np.array(
        [[1 / np.power(10000, 2 * (i // 2) / d_model) for i in range(d_model)]]
    )
    posd = pos * div_term
    pe = np.zeros((max_len, d_model))
    pe[:, 0 : d_model // 2] = np.sin(posd[:, 0::2])
    pe[:, d_model // 2 :] = np.cos(posd[:, 0::2])
    return pe.astype(np.float32)


def _make_embed(flat_n):
    per_w = flat_n // NW
    n_chunks = per_w // CH
    assert n_chunks % 2 == 0

    mesh = plsc.VectorSubcoreMesh(
        core_axis_name="c", subcore_axis_name="s",
        num_cores=NUM_CORES, num_subcores=NUM_SUBCORES,
    )

    @functools.partial(
        pl.kernel,
        out_type=jax.ShapeDtypeStruct((flat_n, D_MODEL), jnp.float32),
        mesh=mesh,
        scratch_types=[
            pltpu.VMEM((n_chunks, CH), jnp.int32),
            pltpu.VMEM((PE_ELEMS,), jnp.float32),
            pltpu.VMEM((2, CH, D_MODEL), jnp.float32),
            pltpu.SemaphoreType.DMA,
            pltpu.SemaphoreType.DMA,
            pltpu.SemaphoreType.DMA,
            pltpu.SemaphoreType.DMA,
        ],
        compiler_params=pltpu.CompilerParams(use_tc_tiling_on_sc=False),
    )
    def embed(x_hbm, table_hbm, pe_hbm, out_hbm, idx_v, pe_v, rows_v, g0, g1, s0, s1):
        wid = lax.axis_index("s") * NUM_CORES + lax.axis_index("c")
        chunk0 = wid * n_chunks
        pltpu.sync_copy(x_hbm.at[pl.ds(chunk0, n_chunks)], idx_v)
        pltpu.sync_copy(pe_hbm, pe_v)

        bufs = (rows_v.at[0], rows_v.at[1])
        gsems = (g0, g1)
        ssems = (s0, s1)

        def gather(g, b):
            return pltpu.make_async_copy(table_hbm.at[idx_v.at[g]], bufs[b], gsems[b])

        def store(g, b):
            return pltpu.make_async_copy(
                bufs[b], out_hbm.at[pl.ds((chunk0 + g) * CH, CH)], ssems[b]
            )

        def compute(g, b):
            pos0 = lax.rem((chunk0 + g) * CH, L_SEQ)
            buf = bufs[b]

            def row_body(r, c2):
                pos = pos0 + r
                pos = jnp.where(pos >= L_SEQ, pos - L_SEQ, pos)
                pb = pos * D_MODEL
                for c in range(D_MODEL // 16):
                    buf[r, pl.ds(c * 16, 16)] = (
                        buf[r, pl.ds(c * 16, 16)] * 8.0
                        + pe_v[pl.ds(pb + c * 16, 16)]
                    )
                return c2

            lax.fori_loop(0, CH, row_body, 0, unroll=8)

        gather(0, 0).start()

        def pair_body(g2, carry):
            for b in range(2):
                g = g2 * 2 + b
                nb = 1 - b

                @pl.when(g + 1 < n_chunks)
                def _():
                    @pl.when(g >= 1)
                    def _():
                        store(g - 1, nb).wait()

                    gather(g + 1, nb).start()

                gather(g, b).wait()
                compute(g, b)
                store(g, b).start()
            return carry

        lax.fori_loop(0, n_chunks // 2, pair_body, 0)
        store(n_chunks - 2, 0).wait()
        store(n_chunks - 1, 1).wait()

    return embed


def kernel(x, emb_table):
    b, l = x.shape
    flat_n = b * l
    assert l == L_SEQ and flat_n % (NW * CH) == 0
    pe = jnp.asarray(_positional_encoding_np(L_SEQ, D_MODEL).reshape(-1))
    x2 = x.reshape(flat_n // CH, CH).astype(jnp.int32)
    out = _make_embed(flat_n)(x2, emb_table, pe)
    return out.reshape(b, l, D_MODEL)


# speedup vs baseline: 1.6131x; 1.1629x over previous
import functools
import numpy as np
import jax
import jax.numpy as jnp
from jax import lax
from jax.experimental import pallas as pl
from jax.experimental.pallas import tpu as pltpu, tpu_sc as plsc

D_MODEL = 64
L_SEQ = 200
PE_ELEMS = L_SEQ * D_MODEL

NUM_CORES = 2
NUM_SUBCORES = 16
NW = NUM_CORES * NUM_SUBCORES

CH = 128


def _positional_encoding_np(max_len, d_model):
    pos = np.expand_dims(np.arange(0, max_len), axis=1)
    div_term = np.array(
        [[1 / np.power(10000, 2 * (i // 2) / d_model) for i in range(d_model)]]
    )
    posd = pos * div_term
    pe = np.zeros((max_len, d_model))
    pe[:, 0 : d_model // 2] = np.sin(posd[:, 0::2])
    pe[:, d_model // 2 :] = np.cos(posd[:, 0::2])
    return pe.astype(np.float32)


def _make_embed(flat_n):
    per_w = flat_n // NW
    n_chunks = per_w // CH
    assert n_chunks % 2 == 0

    mesh = plsc.VectorSubcoreMesh(
        core_axis_name="c", subcore_axis_name="s",
        num_cores=NUM_CORES, num_subcores=NUM_SUBCORES,
    )

    @functools.partial(
        pl.kernel,
        out_type=jax.ShapeDtypeStruct((flat_n, D_MODEL), jnp.float32),
        mesh=mesh,
        scratch_types=[
            pltpu.VMEM((n_chunks, CH), jnp.int32),
            pltpu.VMEM((n_chunks, CH), jnp.int32),
            pltpu.VMEM((PE_ELEMS,), jnp.float32),
            pltpu.VMEM((2, CH, 2 * D_MODEL), jnp.float32),
            pltpu.VMEM((2, CH, D_MODEL), jnp.float32),
            pltpu.SemaphoreType.DMA,
            pltpu.SemaphoreType.DMA,
            pltpu.SemaphoreType.DMA,
            pltpu.SemaphoreType.DMA,
        ],
        compiler_params=pltpu.CompilerParams(use_tc_tiling_on_sc=True),
    )
    def embed(idx_hbm, par_hbm, table_hbm, pe_hbm, out_hbm,
              idx_v, par_v, pe_v, rows_v, sbuf_v, g0, g1, s0, s1):
        wid = lax.axis_index("s") * NUM_CORES + lax.axis_index("c")
        chunk0 = wid * n_chunks
        pltpu.sync_copy(idx_hbm.at[pl.ds(chunk0, n_chunks)], idx_v)
        pltpu.sync_copy(par_hbm.at[pl.ds(chunk0, n_chunks)], par_v)
        pltpu.sync_copy(pe_hbm, pe_v)

        gbufs = (rows_v.at[0], rows_v.at[1])
        sbufs = (sbuf_v.at[0], sbuf_v.at[1])
        gsems = (g0, g1)
        ssems = (s0, s1)

        def gather(g, b):
            return pltpu.make_async_copy(table_hbm.at[idx_v.at[g]], gbufs[b], gsems[b])

        def store(g, b):
            return pltpu.make_async_copy(
                sbufs[b], out_hbm.at[pl.ds((chunk0 + g) * CH, CH)], ssems[b]
            )

        def compute(g, b):
            pos0 = lax.rem((chunk0 + g) * CH, L_SEQ)
            gbuf = gbufs[b]
            sbuf = sbufs[b]

            def grp_body(q, c2):
                r0 = q * 16
                par16 = par_v[g, pl.ds(r0, 16)]
                for k in range(16):
                    r = r0 + k
                    pos = pos0 + r
                    pos = jnp.where(pos >= L_SEQ, pos - L_SEQ, pos)
                    pb = pos * D_MODEL
                    h = par16[k]
                    for c in range(D_MODEL // 16):
                        sbuf[r, pl.ds(c * 16, 16)] = (
                            gbuf[r, pl.ds(h + c * 16, 16)] * 8.0
                            + pe_v[pl.ds(pb + c * 16, 16)]
                        )
                return c2

            lax.fori_loop(0, CH // 16, grp_body, 0)

        gather(0, 0).start()

        def pair_body(g2, carry):
            for b in range(2):
                g = g2 * 2 + b
                nb = 1 - b

                @pl.when(g + 1 < n_chunks)
                def _():
                    gather(g + 1, nb).start()

                @pl.when(g >= 2)
                def _():
                    store(g - 2, b).wait()

                gather(g, b).wait()
                compute(g, b)
                store(g, b).start()
            return carry

        lax.fori_loop(0, n_chunks // 2, pair_body, 0)
        store(n_chunks - 2, 0).wait()
        store(n_chunks - 1, 1).wait()

    return embed


def kernel(x, emb_table):
    b, l = x.shape
    flat_n = b * l
    assert l == L_SEQ and flat_n % (NW * CH) == 0
    pe = jnp.asarray(_positional_encoding_np(L_SEQ, D_MODEL).reshape(-1))
    xf = x.reshape(flat_n // CH, CH).astype(jnp.int32)
    idx2 = xf >> 1
    par = (xf & 1) * D_MODEL
    table2 = emb_table.reshape(emb_table.shape[0] // 2, 2 * D_MODEL)
    out = _make_embed(flat_n)(idx2, par, table2, pe)
    return out.reshape(b, l, D_MODEL)
